# baseline (speedup 1.0000x reference)
import functools

import jax
import jax.numpy as jnp
from jax.experimental import pallas as pl
from jax.experimental.pallas import tpu as pltpu


def _support_body(x_ref, w_ref, o_ref):
    o_ref[...] = jnp.dot(x_ref[...], w_ref[...],
                         preferred_element_type=jnp.float32)


def _gcn_body(adj_ref, sup_ref, b_ref, o_ref):
    acc = jnp.dot(adj_ref[...], sup_ref[...],
                  preferred_element_type=jnp.float32)
    x = acc + b_ref[...]
    o_ref[...] = jnp.where(x > 0, x, jnp.exp(jnp.minimum(x, 0.0)) - 1.0)


def kernel(inputs, adj, weight, bias):
    n, in_f = inputs.shape
    out_f = weight.shape[1]

    bs = 2000
    support = pl.pallas_call(
        _support_body,
        grid=(n // bs,),
        in_specs=[
            pl.BlockSpec((bs, in_f), lambda i: (i, 0)),
            pl.BlockSpec((in_f, out_f), lambda i: (0, 0)),
        ],
        out_specs=pl.BlockSpec((bs, out_f), lambda i: (i, 0)),
        out_shape=jax.ShapeDtypeStruct((n, out_f), jnp.float32),
    )(inputs, weight)

    bm = 400
    bias2d = bias.reshape(1, out_f)
    out = pl.pallas_call(
        _gcn_body,
        grid=(n // bm,),
        in_specs=[
            pl.BlockSpec((bm, n), lambda m: (m, 0)),
            pl.BlockSpec((n, out_f), lambda m: (0, 0)),
            pl.BlockSpec((1, out_f), lambda m: (0, 0)),
        ],
        out_specs=pl.BlockSpec((bm, out_f), lambda m: (m, 0)),
        out_shape=jax.ShapeDtypeStruct((n, out_f), jnp.float32),
        compiler_params=pltpu.CompilerParams(
            dimension_semantics=("parallel",),
        ),
    )(adj, support, bias2d)
    return out


# speedup vs baseline: 1.0578x; 1.0578x over previous
import jax
import jax.numpy as jnp
from jax.experimental import pallas as pl
from jax.experimental.pallas import tpu as pltpu


def _gcn_body(x_ref, w_ref, b_ref, adj_ref, o_ref, sup_ref):
    @pl.when(pl.program_id(0) == 0)
    def _build_support():
        sup_ref[...] = jnp.dot(x_ref[...], w_ref[...],
                               preferred_element_type=jnp.float32)

    acc = jnp.dot(adj_ref[...], sup_ref[...],
                  preferred_element_type=jnp.float32)
    x = acc + b_ref[...]
    o_ref[...] = jnp.where(x > 0, x, jnp.exp(jnp.minimum(x, 0.0)) - 1.0)


def kernel(inputs, adj, weight, bias):
    n, in_f = inputs.shape
    out_f = weight.shape[1]
    bm = 400
    bias2d = bias.reshape(1, out_f)
    out = pl.pallas_call(
        _gcn_body,
        grid=(n // bm,),
        in_specs=[
            pl.BlockSpec((n, in_f), lambda m: (0, 0)),
            pl.BlockSpec((in_f, out_f), lambda m: (0, 0)),
            pl.BlockSpec((1, out_f), lambda m: (0, 0)),
            pl.BlockSpec((bm, n), lambda m: (m, 0)),
        ],
        out_specs=pl.BlockSpec((bm, out_f), lambda m: (m, 0)),
        out_shape=jax.ShapeDtypeStruct((n, out_f), jnp.float32),
        scratch_shapes=[pltpu.VMEM((n, out_f), jnp.float32)],
        compiler_params=pltpu.CompilerParams(
            dimension_semantics=("arbitrary",),
        ),
    )(inputs, weight, bias2d, adj)
    return out
